# Initial kernel scaffold; baseline (speedup 1.0000x reference)
import jax
import jax.numpy as jnp
from jax.experimental import pallas as pl


def kernel(x, edge_index, edge_attr, batch, W_root0, W_rel0, b0, W_root_rest, W_rel_rest, b_rest, fc1_w, fc1_b, fc2_w, fc2_b):
    raise NotImplementedError("write your pallas kernel here")


# speedup vs baseline: 6.2892x; 6.2892x over previous
import functools

import jax
import jax.numpy as jnp
from jax import lax
from jax.experimental import pallas as pl
from jax.experimental.pallas import tpu as pltpu
from jax.experimental.pallas import tpu_sc as plsc

N = 10000
E = 640000
R = 4
G = 64
L = 16
NC = 2
NS = 16
CH = 80
EPW = E // (NC * NS)
EPT = E // NS
ROWS4N = R * N
CROWS = 2560
RPT = CROWS // NS
NPT = N // NS
NP = 10240
NBLK = NP // 128

_SC_MESH = dict(core_axis_name="c", subcore_axis_name="s", num_cores=NC,
                num_subcores=NS)


@functools.partial(
    pl.kernel,
    out_type=jax.ShapeDtypeStruct((NC * CROWS, L), jnp.float32),
    mesh=plsc.VectorSubcoreMesh(**_SC_MESH),
    compiler_params=pltpu.CompilerParams(needs_layout_passes=False, use_tc_tiling_on_sc=False),
    scratch_types=[
        pltpu.VMEM_SHARED((CROWS, L), jnp.float32),
        pltpu.VMEM((CH,), jnp.int32),
        pltpu.VMEM((CH,), jnp.int32),
        pltpu.VMEM((CH,), jnp.int32),
        pltpu.VMEM((CH,), jnp.int32),
        pltpu.VMEM((CH, L), jnp.float32),
        pltpu.VMEM((RPT, L), jnp.float32),
    ],
)
def _count_kernel(dst_hbm, rel_hbm, cnt_hbm, cacc, dstb, relb, rowb, laneb,
                  ohb, zb):
    c = lax.axis_index("c")
    sid = lax.axis_index("s")
    wid = sid * NC + c
    zero = jnp.zeros((L,), jnp.float32)

    @pl.loop(0, RPT)
    def _(i):
        zb[i, :] = zero

    pltpu.sync_copy(zb, cacc.at[pl.ds(sid * RPT, RPT), :])
    plsc.subcore_barrier()

    @pl.loop(0, EPW // CH)
    def _(ci):
        base = wid * EPW + ci * CH
        pltpu.sync_copy(dst_hbm.at[pl.ds(base, CH)], dstb)
        pltpu.sync_copy(rel_hbm.at[pl.ds(base, CH)], relb)

        for j in range(CH // L):
            sl = pl.ds(j * L, L)
            idx = relb[sl] * N + dstb[sl]
            rowb[sl] = lax.shift_right_logical(idx, 4)
            laneb[sl] = lax.bitwise_and(idx, L - 1)

        @pl.loop(0, CH)
        def _(i):
            lane_s = plsc.load_gather(laneb, [jnp.full((L,), i, jnp.int32)])
            ohb[i, :] = jnp.where(lax.iota(jnp.int32, L) == lane_s, 1.0, 0.0)

        pltpu.sync_copy(ohb, cacc.at[rowb], add=True)

    plsc.subcore_barrier()
    pltpu.sync_copy(cacc.at[pl.ds(sid * RPT, RPT), :],
                    cnt_hbm.at[pl.ds(c * CROWS + sid * RPT, RPT), :])


@functools.partial(
    pl.kernel,
    out_type=(jax.ShapeDtypeStruct((E,), jnp.float32),
              jax.ShapeDtypeStruct((E,), jnp.int32)),
    mesh=plsc.VectorSubcoreMesh(**_SC_MESH),
    compiler_params=pltpu.CompilerParams(needs_layout_passes=False, use_tc_tiling_on_sc=False),
    scratch_types=[
        pltpu.VMEM((CROWS, L), jnp.float32),
        pltpu.VMEM((CROWS, L), jnp.float32),
        pltpu.VMEM((CH,), jnp.int32),
        pltpu.VMEM((CH,), jnp.int32),
        pltpu.VMEM((CH,), jnp.int32),
        pltpu.VMEM((CH,), jnp.float32),
        pltpu.VMEM((CH,), jnp.int32),
    ],
)
def _scale_kernel(src_hbm, dst_hbm, rel_hbm, cnt_hbm, scale_hbm, gidx_hbm,
                  c0, c1, srcb, dstb, relb, sclb, gb):
    c = lax.axis_index("c")
    sid = lax.axis_index("s")
    wid = sid * NC + c
    pltpu.sync_copy(cnt_hbm.at[pl.ds(0, CROWS), :], c0)
    pltpu.sync_copy(cnt_hbm.at[pl.ds(CROWS, CROWS), :], c1)

    @pl.loop(0, EPW // CH)
    def _(ci):
        base = wid * EPW + ci * CH
        pltpu.sync_copy(src_hbm.at[pl.ds(base, CH)], srcb)
        pltpu.sync_copy(dst_hbm.at[pl.ds(base, CH)], dstb)
        pltpu.sync_copy(rel_hbm.at[pl.ds(base, CH)], relb)
        for j in range(CH // L):
            sl = pl.ds(j * L, L)
            r16 = relb[sl]
            idx = r16 * N + dstb[sl]
            row = lax.shift_right_logical(idx, 4)
            lane = lax.bitwise_and(idx, L - 1)
            cnt = (plsc.load_gather(c0, [row, lane])
                   + plsc.load_gather(c1, [row, lane]))
            sclb[sl] = 1.0 / jnp.maximum(cnt, 1.0)
            gb[sl] = r16 * N + srcb[sl]
        pltpu.sync_copy(sclb, scale_hbm.at[pl.ds(base, CH)])
        pltpu.sync_copy(gb, gidx_hbm.at[pl.ds(base, CH)])


@functools.partial(
    pl.kernel,
    out_type=jax.ShapeDtypeStruct((NC * N, 128), jnp.float32),
    mesh=plsc.VectorSubcoreMesh(**_SC_MESH),
    compiler_params=pltpu.CompilerParams(needs_layout_passes=False, use_tc_tiling_on_sc=False),
    scratch_types=[
        pltpu.VMEM_SHARED((N, 128), jnp.float32),
        pltpu.VMEM((CH,), jnp.int32),
        pltpu.VMEM((CH,), jnp.int32),
        pltpu.VMEM((CH,), jnp.float32),
        pltpu.VMEM((CH, 128), jnp.float32),
        pltpu.VMEM((125, 128), jnp.float32),
        pltpu.SemaphoreType.DMA,
    ],
)
def _edge_kernel(table_hbm, gidx_hbm, dst_hbm, scale_hbm, msg_hbm,
                 acc, idxb, dstb, sclb, rows, zb, gsem):
    c = lax.axis_index("c")
    sid = lax.axis_index("s")
    zero = jnp.zeros((L,), jnp.float32)

    @pl.loop(0, 125)
    def _(i):
        for k in range(8):
            zb[i, pl.ds(k * L, L)] = zero

    for j in range(5):
        pltpu.sync_copy(zb, acc.at[pl.ds(sid * NPT + j * 125, 125), :])
    plsc.subcore_barrier()

    coff = jnp.full((L,), c * ROWS4N, jnp.int32)

    @pl.loop(0, EPT // CH)
    def _(ci):
        base = sid * EPT + ci * CH
        pltpu.sync_copy(gidx_hbm.at[pl.ds(base, CH)], idxb)
        pltpu.sync_copy(dst_hbm.at[pl.ds(base, CH)], dstb)
        pltpu.sync_copy(scale_hbm.at[pl.ds(base, CH)], sclb)
        for j in range(CH // L):
            sl = pl.ds(j * L, L)
            idxb[sl] = idxb[sl] + coff
        pltpu.async_copy(table_hbm.at[idxb], rows, gsem).wait()

        @pl.loop(0, CH)
        def _(i):
            s = plsc.load_gather(sclb, [jnp.full((L,), i, jnp.int32)])
            for k in range(8):
                sl = pl.ds(k * L, L)
                rows[i, sl] = rows[i, sl] * s

        pltpu.sync_copy(rows, acc.at[dstb], add=True)

    plsc.subcore_barrier()
    for j in range(5):
        pltpu.sync_copy(acc.at[pl.ds(sid * NPT + j * 125, 125), :],
                        msg_hbm.at[pl.ds(c * N + sid * NPT + j * 125, 125), :])


def _mm_kernel(root, msg, x, w2, b2, has_prev, kdim):
    bn = 1000
    grid = (N // bn,)

    def body(*refs):
        if has_prev:
            root_ref, msg_ref, w2_ref, b2_ref, trel_ref, troot_ref = refs
            h = jnp.concatenate(
                [jnp.maximum(root_ref[0] + msg_ref[0], 0.0),
                 jnp.maximum(root_ref[1] + msg_ref[1], 0.0)], axis=1)
        else:
            x_ref, w2_ref, b2_ref, trel_ref, troot_ref = refs
            h = x_ref[...]
        for j in range(8):
            trel_ref[j] = jnp.dot(h, w2_ref[:, j * 128:(j + 1) * 128],
                                  preferred_element_type=jnp.float32)
        for j in range(2):
            troot_ref[j] = (jnp.dot(h, w2_ref[:, (8 + j) * 128:(9 + j) * 128],
                                    preferred_element_type=jnp.float32)
                            + b2_ref[pl.ds(j, 1), :])

    if has_prev:
        in_specs = [
            pl.BlockSpec((NC, bn, 128), lambda n: (0, n, 0)),
            pl.BlockSpec((NC, bn, 128), lambda n: (0, n, 0)),
            pl.BlockSpec((kdim, 1280), lambda n: (0, 0)),
            pl.BlockSpec((NC, 128), lambda n: (0, 0)),
        ]
        args = (root, msg, w2, b2)
    else:
        in_specs = [
            pl.BlockSpec((bn, kdim), lambda n: (n, 0)),
            pl.BlockSpec((kdim, 1280), lambda n: (0, 0)),
            pl.BlockSpec((NC, 128), lambda n: (0, 0)),
        ]
        args = (x, w2, b2)

    return pl.pallas_call(
        body,
        grid=grid,
        in_specs=in_specs,
        out_specs=[
            pl.BlockSpec((8, bn, 128), lambda n: (0, n, 0)),
            pl.BlockSpec((NC, bn, 128), lambda n: (0, n, 0)),
        ],
        out_shape=[
            jax.ShapeDtypeStruct((8, N, 128), jnp.float32),
            jax.ShapeDtypeStruct((NC, N, 128), jnp.float32),
        ],
    )(*args)


def _pool_kernel(root, msg, batch3, w1, b1, w2, b2):
    def body(root_ref, msg_ref, batch_ref, w1_ref, b1_ref, w2_ref, b2_ref,
             out_ref, sum_s, max_s, cnt_s):
        i = pl.program_id(0)

        @pl.when(i == 0)
        def _():
            sum_s[...] = jnp.zeros_like(sum_s)
            max_s[...] = jnp.zeros_like(max_s)
            cnt_s[...] = jnp.zeros_like(cnt_s)

        h = jnp.concatenate(
            [jnp.maximum(root_ref[0] + msg_ref[0], 0.0),
             jnp.maximum(root_ref[1] + msg_ref[1], 0.0)], axis=1)
        bt = batch_ref[0]
        giota = lax.broadcasted_iota(jnp.int32, (G, 128), 0)
        ohT = (jnp.broadcast_to(bt, (G, 128)) == giota).astype(jnp.float32)
        sum_s[...] += jnp.dot(ohT, h, preferred_element_type=jnp.float32)
        cnt_s[...] += ohT
        ident = (lax.broadcasted_iota(jnp.int32, (128, 128), 0)
                 == lax.broadcasted_iota(jnp.int32, (128, 128), 1)
                 ).astype(jnp.float32)
        btcol = lax.dot_general(ident, bt.astype(jnp.float32),
                                (((1,), (1,)), ((), ())),
                                preferred_element_type=jnp.float32)
        for g in range(G):
            mask = (btcol == float(g)).astype(jnp.float32)
            mg = jnp.max(h * mask, axis=0, keepdims=True)
            max_s[pl.ds(g, 1), :] = jnp.maximum(max_s[pl.ds(g, 1), :], mg)

        @pl.when(i == NBLK - 1)
        def _():
            cnt = jnp.sum(cnt_s[...], axis=1, keepdims=True)
            mean = sum_s[...] / jnp.maximum(cnt, 1.0)
            gfeat = jnp.concatenate([mean, max_s[...]], axis=1)
            z = jnp.maximum(
                jnp.dot(gfeat, w1_ref[...], preferred_element_type=jnp.float32)
                + b1_ref[...], 0.0)
            o = (jnp.dot(z, w2_ref[...], preferred_element_type=jnp.float32)
                 + b2_ref[...])
            out_ref[...] = 1.0 / (1.0 + jnp.exp(-o))

    return pl.pallas_call(
        body,
        grid=(NBLK,),
        in_specs=[
            pl.BlockSpec((NC, 128, 128), lambda n: (0, n, 0)),
            pl.BlockSpec((NC, 128, 128), lambda n: (0, n, 0)),
            pl.BlockSpec((1, 1, 128), lambda n: (n, 0, 0)),
            pl.BlockSpec((512, 128), lambda n: (0, 0)),
            pl.BlockSpec((1, 128), lambda n: (0, 0)),
            pl.BlockSpec((128, 1), lambda n: (0, 0)),
            pl.BlockSpec((1, 1), lambda n: (0, 0)),
        ],
        out_specs=pl.BlockSpec((G, 1), lambda n: (0, 0)),
        out_shape=jax.ShapeDtypeStruct((G, 1), jnp.float32),
        scratch_shapes=[
            pltpu.VMEM((G, 256), jnp.float32),
            pltpu.VMEM((G, 256), jnp.float32),
            pltpu.VMEM((G, 128), jnp.float32),
        ],
    )(root, msg, batch3, w1, b1, w2, b2)


def _build_w2(w_rel, w_root):
    cols = [w_rel[r, :, s * 128:(s + 1) * 128] for s in range(NC)
            for r in range(R)]
    cols += [w_root[:, s * 128:(s + 1) * 128] for s in range(NC)]
    return jnp.concatenate(cols, axis=1)


def kernel(x, edge_index, edge_attr, batch, W_root0, W_rel0, b0,
           W_root_rest, W_rel_rest, b_rest, fc1_w, fc1_b, fc2_w, fc2_b):
    src = edge_index[0]
    dst = edge_index[1]
    rel = edge_attr

    cnt = _count_kernel(dst, rel)
    scale, gidx = _scale_kernel(src, dst, rel, cnt)

    trel, troot = _mm_kernel(None, None, x, _build_w2(W_rel0, W_root0),
                             b0.reshape(NC, 128), False, 128)
    msg = _edge_kernel(trel.reshape(8 * N, 128), gidx, dst, scale)
    root, msg = troot, msg.reshape(NC, N, 128)

    for i in range(5):
        w2 = _build_w2(W_rel_rest[i], W_root_rest[i])
        trel, troot = _mm_kernel(root, msg, None, w2,
                                 b_rest[i].reshape(NC, 128), True, 256)
        msg = _edge_kernel(trel.reshape(8 * N, 128), gidx, dst, scale)
        root, msg = troot, msg.reshape(NC, N, 128)

    rootp = jnp.pad(root, ((0, 0), (0, NP - N), (0, 0)))
    msgp = jnp.pad(msg, ((0, 0), (0, NP - N), (0, 0)))
    batchp = jnp.pad(batch, (0, NP - N),
                     constant_values=G).reshape(NBLK, 1, 128)
    out = _pool_kernel(rootp, msgp, batchp, fc1_w, fc1_b.reshape(1, 128),
                       fc2_w, fc2_b.reshape(1, 1))
    return out.reshape(G)


# speedup vs baseline: 13.8140x; 2.1965x over previous
import functools

import jax
import jax.numpy as jnp
from jax import lax
from jax.experimental import pallas as pl
from jax.experimental.pallas import tpu as pltpu
from jax.experimental.pallas import tpu_sc as plsc

N = 10000
E = 640000
R = 4
G = 64
L = 16
NC = 2
NS = 16
CH = 80
EPW = E // (NC * NS)
EPT = E // NS
ROWS4N = R * N
CROWS = 2560
RPT = CROWS // NS
NPT = N // NS
NP = 10240
NBLK = NP // 128

_SC_MESH = dict(core_axis_name="c", subcore_axis_name="s", num_cores=NC,
                num_subcores=NS)


@functools.partial(
    pl.kernel,
    out_type=jax.ShapeDtypeStruct((NC * CROWS, L), jnp.float32),
    mesh=plsc.VectorSubcoreMesh(**_SC_MESH),
    compiler_params=pltpu.CompilerParams(needs_layout_passes=False, use_tc_tiling_on_sc=False),
    scratch_types=[
        pltpu.VMEM_SHARED((CROWS, L), jnp.float32),
        pltpu.VMEM((CH,), jnp.int32),
        pltpu.VMEM((CH,), jnp.int32),
        pltpu.VMEM((CH,), jnp.int32),
        pltpu.VMEM((CH,), jnp.int32),
        pltpu.VMEM((CH, L), jnp.float32),
        pltpu.VMEM((RPT, L), jnp.float32),
    ],
)
def _count_kernel(dst_hbm, rel_hbm, cnt_hbm, cacc, dstb, relb, rowb, laneb,
                  ohb, zb):
    c = lax.axis_index("c")
    sid = lax.axis_index("s")
    wid = sid * NC + c
    zero = jnp.zeros((L,), jnp.float32)

    @pl.loop(0, RPT)
    def _(i):
        zb[i, :] = zero

    pltpu.sync_copy(zb, cacc.at[pl.ds(sid * RPT, RPT), :])
    plsc.subcore_barrier()

    @pl.loop(0, EPW // CH)
    def _(ci):
        base = wid * EPW + ci * CH
        pltpu.sync_copy(dst_hbm.at[pl.ds(base, CH)], dstb)
        pltpu.sync_copy(rel_hbm.at[pl.ds(base, CH)], relb)

        for j in range(CH // L):
            sl = pl.ds(j * L, L)
            idx = relb[sl] * N + dstb[sl]
            rowb[sl] = lax.shift_right_logical(idx, 4)
            laneb[sl] = lax.bitwise_and(idx, L - 1)

        @pl.loop(0, CH)
        def _(i):
            lane_s = plsc.load_gather(laneb, [jnp.full((L,), i, jnp.int32)])
            ohb[i, :] = jnp.where(lax.iota(jnp.int32, L) == lane_s, 1.0, 0.0)

        pltpu.sync_copy(ohb, cacc.at[rowb], add=True)

    plsc.subcore_barrier()
    pltpu.sync_copy(cacc.at[pl.ds(sid * RPT, RPT), :],
                    cnt_hbm.at[pl.ds(c * CROWS + sid * RPT, RPT), :])


@functools.partial(
    pl.kernel,
    out_type=(jax.ShapeDtypeStruct((E,), jnp.float32),
              jax.ShapeDtypeStruct((E,), jnp.int32)),
    mesh=plsc.VectorSubcoreMesh(**_SC_MESH),
    compiler_params=pltpu.CompilerParams(needs_layout_passes=False, use_tc_tiling_on_sc=False),
    scratch_types=[
        pltpu.VMEM((CROWS, L), jnp.float32),
        pltpu.VMEM((CROWS, L), jnp.float32),
        pltpu.VMEM((CH,), jnp.int32),
        pltpu.VMEM((CH,), jnp.int32),
        pltpu.VMEM((CH,), jnp.int32),
        pltpu.VMEM((CH,), jnp.float32),
        pltpu.VMEM((CH,), jnp.int32),
    ],
)
def _scale_kernel(src_hbm, dst_hbm, rel_hbm, cnt_hbm, scale_hbm, gidx_hbm,
                  c0, c1, srcb, dstb, relb, sclb, gb):
    c = lax.axis_index("c")
    sid = lax.axis_index("s")
    wid = sid * NC + c
    pltpu.sync_copy(cnt_hbm.at[pl.ds(0, CROWS), :], c0)
    pltpu.sync_copy(cnt_hbm.at[pl.ds(CROWS, CROWS), :], c1)

    @pl.loop(0, EPW // CH)
    def _(ci):
        base = wid * EPW + ci * CH
        pltpu.sync_copy(src_hbm.at[pl.ds(base, CH)], srcb)
        pltpu.sync_copy(dst_hbm.at[pl.ds(base, CH)], dstb)
        pltpu.sync_copy(rel_hbm.at[pl.ds(base, CH)], relb)
        for j in range(CH // L):
            sl = pl.ds(j * L, L)
            r16 = relb[sl]
            idx = r16 * N + dstb[sl]
            row = lax.shift_right_logical(idx, 4)
            lane = lax.bitwise_and(idx, L - 1)
            cnt = (plsc.load_gather(c0, [row, lane])
                   + plsc.load_gather(c1, [row, lane]))
            sclb[sl] = 1.0 / jnp.maximum(cnt, 1.0)
            gb[sl] = r16 * N + srcb[sl]
        pltpu.sync_copy(sclb, scale_hbm.at[pl.ds(base, CH)])
        pltpu.sync_copy(gb, gidx_hbm.at[pl.ds(base, CH)])


NSUB = 10
NCHK = EPT // CH
CPR = E // CH


@functools.partial(
    pl.kernel,
    out_type=jax.ShapeDtypeStruct((NC * N, 128), jnp.float32),
    mesh=plsc.VectorSubcoreMesh(**_SC_MESH),
    compiler_params=pltpu.CompilerParams(needs_layout_passes=False, use_tc_tiling_on_sc=False),
    scratch_types=[
        pltpu.VMEM_SHARED((N, 128), jnp.float32),
        pltpu.VMEM((2 * NSUB, CH), jnp.int32),
        pltpu.VMEM((2 * NSUB, CH), jnp.int32),
        pltpu.VMEM((2 * NSUB, CH), jnp.float32),
        pltpu.VMEM((CH, 128), jnp.float32),
        pltpu.VMEM((CH, 128), jnp.float32),
        pltpu.VMEM((125, 128), jnp.float32),
        pltpu.SemaphoreType.DMA,
        pltpu.SemaphoreType.DMA,
        pltpu.SemaphoreType.DMA,
        pltpu.SemaphoreType.DMA,
    ],
)
def _edge_kernel(table_hbm, gidx_hbm, dst_hbm, scale_hbm, msg_hbm,
                 acc, idxb, dstb, sclb, rows0, rows1, zb,
                 gsem0, gsem1, ssem0, ssem1):
    c = lax.axis_index("c")
    sid = lax.axis_index("s")
    zero = jnp.zeros((L,), jnp.float32)
    rows = (rows0, rows1)
    gsem = (gsem0, gsem1)
    ssem = (ssem0, ssem1)

    @pl.loop(0, 125)
    def _(i):
        for k in range(8):
            zb[i, pl.ds(k * L, L)] = zero

    for j in range(5):
        pltpu.sync_copy(zb, acc.at[pl.ds(sid * NPT + j * 125, 125), :])
    plsc.subcore_barrier()

    coff = jnp.full((L,), c * ROWS4N, jnp.int32)

    def load_meta(si, slot):
        crow = sid * NCHK + si * NSUB
        mrow = slot * NSUB
        pltpu.sync_copy(gidx_hbm.at[pl.ds(crow, NSUB), :],
                        idxb.at[pl.ds(mrow, NSUB), :])
        pltpu.sync_copy(dst_hbm.at[pl.ds(crow, NSUB), :],
                        dstb.at[pl.ds(mrow, NSUB), :])
        pltpu.sync_copy(scale_hbm.at[pl.ds(crow, NSUB), :],
                        sclb.at[pl.ds(mrow, NSUB), :])

        @pl.loop(0, NSUB)
        def _(r):
            for j in range(CH // L):
                sl = pl.ds(j * L, L)
                idxb[mrow + r, sl] = idxb[mrow + r, sl] + coff

    def fire_gather(ci, b):
        mr = (ci // NSUB) % 2 * NSUB + ci % NSUB
        return pltpu.async_copy(table_hbm.at[idxb.at[mr]], rows[b], gsem[b])

    def fire_scatter(ci, b):
        mr = (ci // NSUB) % 2 * NSUB + ci % NSUB
        return pltpu.async_copy(rows[b], acc.at[dstb.at[mr]], ssem[b],
                                add=True)

    load_meta(0, 0)
    fire_gather(0, 0)

    def step(ci, b):
        sub = ci % NSUB

        @pl.when(ci >= 1)
        def _():
            pltpu.make_async_copy(rows[1 - b], acc.at[dstb.at[0]],
                                  ssem[1 - b]).wait()

        @pl.when(jnp.logical_and(sub == NSUB - 1, ci + 1 < NCHK))
        def _():
            load_meta(ci // NSUB + 1, (ci // NSUB + 1) % 2)

        @pl.when(ci + 1 < NCHK)
        def _():
            fire_gather(ci + 1, 1 - b)

        mr = (ci // NSUB) % 2 * NSUB + sub
        pltpu.make_async_copy(table_hbm.at[idxb.at[mr]], rows[b],
                              gsem[b]).wait()
        rsp = jnp.full((L,), mr, jnp.int32)

        @pl.loop(0, CH)
        def _(i):
            s = plsc.load_gather(sclb, [rsp, jnp.full((L,), i, jnp.int32)])
            for k in range(8):
                sl = pl.ds(k * L, L)
                rows[b][i, sl] = rows[b][i, sl] * s

        fire_scatter(ci, b)

    @pl.loop(0, NCHK // 2)
    def _(pi):
        step(2 * pi, 0)
        step(2 * pi + 1, 1)

    pltpu.make_async_copy(rows[1], acc.at[dstb.at[0]], ssem[1]).wait()

    plsc.subcore_barrier()
    for j in range(5):
        pltpu.sync_copy(acc.at[pl.ds(sid * NPT + j * 125, 125), :],
                        msg_hbm.at[pl.ds(c * N + sid * NPT + j * 125, 125), :])


def _mm_kernel(root, msg, x, w2, b2, has_prev, kdim):
    bn = 1000
    grid = (N // bn,)

    def body(*refs):
        if has_prev:
            root_ref, msg_ref, w2_ref, b2_ref, trel_ref, troot_ref = refs
            h = jnp.concatenate(
                [jnp.maximum(root_ref[0] + msg_ref[0], 0.0),
                 jnp.maximum(root_ref[1] + msg_ref[1], 0.0)], axis=1)
        else:
            x_ref, w2_ref, b2_ref, trel_ref, troot_ref = refs
            h = x_ref[...]
        for j in range(8):
            trel_ref[j] = jnp.dot(h, w2_ref[:, j * 128:(j + 1) * 128],
                                  preferred_element_type=jnp.float32)
        for j in range(2):
            troot_ref[j] = (jnp.dot(h, w2_ref[:, (8 + j) * 128:(9 + j) * 128],
                                    preferred_element_type=jnp.float32)
                            + b2_ref[pl.ds(j, 1), :])

    if has_prev:
        in_specs = [
            pl.BlockSpec((NC, bn, 128), lambda n: (0, n, 0)),
            pl.BlockSpec((NC, bn, 128), lambda n: (0, n, 0)),
            pl.BlockSpec((kdim, 1280), lambda n: (0, 0)),
            pl.BlockSpec((NC, 128), lambda n: (0, 0)),
        ]
        args = (root, msg, w2, b2)
    else:
        in_specs = [
            pl.BlockSpec((bn, kdim), lambda n: (n, 0)),
            pl.BlockSpec((kdim, 1280), lambda n: (0, 0)),
            pl.BlockSpec((NC, 128), lambda n: (0, 0)),
        ]
        args = (x, w2, b2)

    return pl.pallas_call(
        body,
        grid=grid,
        in_specs=in_specs,
        out_specs=[
            pl.BlockSpec((8, bn, 128), lambda n: (0, n, 0)),
            pl.BlockSpec((NC, bn, 128), lambda n: (0, n, 0)),
        ],
        out_shape=[
            jax.ShapeDtypeStruct((8, N, 128), jnp.float32),
            jax.ShapeDtypeStruct((NC, N, 128), jnp.float32),
        ],
    )(*args)


def _pool_kernel(root, msg, batch3, w1, b1, w2, b2):
    def body(root_ref, msg_ref, batch_ref, w1_ref, b1_ref, w2_ref, b2_ref,
             out_ref, sum_s, max_s, cnt_s):
        i = pl.program_id(0)

        @pl.when(i == 0)
        def _():
            sum_s[...] = jnp.zeros_like(sum_s)
            max_s[...] = jnp.zeros_like(max_s)
            cnt_s[...] = jnp.zeros_like(cnt_s)

        h = jnp.concatenate(
            [jnp.maximum(root_ref[0] + msg_ref[0], 0.0),
             jnp.maximum(root_ref[1] + msg_ref[1], 0.0)], axis=1)
        bt = batch_ref[0]
        giota = lax.broadcasted_iota(jnp.int32, (G, 128), 0)
        ohT = (jnp.broadcast_to(bt, (G, 128)) == giota).astype(jnp.float32)
        sum_s[...] += jnp.dot(ohT, h, preferred_element_type=jnp.float32)
        cnt_s[...] += ohT
        ident = (lax.broadcasted_iota(jnp.int32, (128, 128), 0)
                 == lax.broadcasted_iota(jnp.int32, (128, 128), 1)
                 ).astype(jnp.float32)
        btcol = lax.dot_general(ident, bt.astype(jnp.float32),
                                (((1,), (1,)), ((), ())),
                                preferred_element_type=jnp.float32)
        for g in range(G):
            mask = (btcol == float(g)).astype(jnp.float32)
            mg = jnp.max(h * mask, axis=0, keepdims=True)
            max_s[pl.ds(g, 1), :] = jnp.maximum(max_s[pl.ds(g, 1), :], mg)

        @pl.when(i == NBLK - 1)
        def _():
            cnt = jnp.sum(cnt_s[...], axis=1, keepdims=True)
            mean = sum_s[...] / jnp.maximum(cnt, 1.0)
            gfeat = jnp.concatenate([mean, max_s[...]], axis=1)
            z = jnp.maximum(
                jnp.dot(gfeat, w1_ref[...], preferred_element_type=jnp.float32)
                + b1_ref[...], 0.0)
            o = (jnp.dot(z, w2_ref[...], preferred_element_type=jnp.float32)
                 + b2_ref[...])
            out_ref[...] = 1.0 / (1.0 + jnp.exp(-o))

    return pl.pallas_call(
        body,
        grid=(NBLK,),
        in_specs=[
            pl.BlockSpec((NC, 128, 128), lambda n: (0, n, 0)),
            pl.BlockSpec((NC, 128, 128), lambda n: (0, n, 0)),
            pl.BlockSpec((1, 1, 128), lambda n: (n, 0, 0)),
            pl.BlockSpec((512, 128), lambda n: (0, 0)),
            pl.BlockSpec((1, 128), lambda n: (0, 0)),
            pl.BlockSpec((128, 1), lambda n: (0, 0)),
            pl.BlockSpec((1, 1), lambda n: (0, 0)),
        ],
        out_specs=pl.BlockSpec((G, 1), lambda n: (0, 0)),
        out_shape=jax.ShapeDtypeStruct((G, 1), jnp.float32),
        scratch_shapes=[
            pltpu.VMEM((G, 256), jnp.float32),
            pltpu.VMEM((G, 256), jnp.float32),
            pltpu.VMEM((G, 128), jnp.float32),
        ],
    )(root, msg, batch3, w1, b1, w2, b2)


def _build_w2(w_rel, w_root):
    cols = [w_rel[r, :, s * 128:(s + 1) * 128] for s in range(NC)
            for r in range(R)]
    cols += [w_root[:, s * 128:(s + 1) * 128] for s in range(NC)]
    return jnp.concatenate(cols, axis=1)


def kernel(x, edge_index, edge_attr, batch, W_root0, W_rel0, b0,
           W_root_rest, W_rel_rest, b_rest, fc1_w, fc1_b, fc2_w, fc2_b):
    src = edge_index[0]
    dst = edge_index[1]
    rel = edge_attr

    cnt = _count_kernel(dst, rel)
    scale, gidx = _scale_kernel(src, dst, rel, cnt)
    scale2 = scale.reshape(CPR, CH)
    gidx2 = gidx.reshape(CPR, CH)
    dst2 = dst.reshape(CPR, CH)

    trel, troot = _mm_kernel(None, None, x, _build_w2(W_rel0, W_root0),
                             b0.reshape(NC, 128), False, 128)
    msg = _edge_kernel(trel.reshape(8 * N, 128), gidx2, dst2, scale2)
    root, msg = troot, msg.reshape(NC, N, 128)

    for i in range(5):
        w2 = _build_w2(W_rel_rest[i], W_root_rest[i])
        trel, troot = _mm_kernel(root, msg, None, w2,
                                 b_rest[i].reshape(NC, 128), True, 256)
        msg = _edge_kernel(trel.reshape(8 * N, 128), gidx2, dst2, scale2)
        root, msg = troot, msg.reshape(NC, N, 128)

    rootp = jnp.pad(root, ((0, 0), (0, NP - N), (0, 0)))
    msgp = jnp.pad(msg, ((0, 0), (0, NP - N), (0, 0)))
    batchp = jnp.pad(batch, (0, NP - N),
                     constant_values=G).reshape(NBLK, 1, 128)
    out = _pool_kernel(rootp, msgp, batchp, fc1_w, fc1_b.reshape(1, 128),
                       fc2_w, fc2_b.reshape(1, 1))
    return out.reshape(G)
